# baseline (speedup 1.0000x reference)
import functools

import jax
import jax.numpy as jnp
from jax import lax
from jax.experimental import pallas as pl
from jax.experimental.pallas import tpu as pltpu
from jax.experimental.pallas import tpu_sc as plsc

N = 10000
NP = 10240
F1 = 128
C = 40
F2 = 48

_NC, _NS = 2, 16
_NW = _NC * _NS
_K = 80
_GR = 128
_NB = 4
_GA = 2
_DW = 16
_DR = NP // 32
_BN = 2048


def _geom(rows, passes, gr):
    if passes == 1 and gr == 0:
        rng = rows
    else:
        rng = -(-rows // (passes * 128)) * 128
    lens = [min(rng, rows - i * rng) for i in range(passes)]
    ar = rng + gr
    ztiles = _NS
    zpt = ar // ztiles
    while zpt % 8:
        ztiles //= 2
        zpt = ar // ztiles
    return rng, lens, ar, ztiles, zpt


def _make_prop(nch, passes, rows=NP, gr=_GR):
    rng, lens, ar, ztiles, zpt = _geom(rows, passes, gr)
    mesh = plsc.VectorSubcoreMesh(core_axis_name="c", subcore_axis_name="s")

    @functools.partial(
        pl.kernel,
        out_type=jax.ShapeDtypeStruct((_NC, rows, F1), jnp.float32),
        mesh=mesh,
        scratch_types=[
            pltpu.VMEM((nch, _K), jnp.int32),
            pltpu.VMEM((nch, _K), jnp.int32),
        ] + [pltpu.VMEM((_K,), jnp.int32) for _ in range(_NB)]
        + [pltpu.VMEM((_K, F1), jnp.float32) for _ in range(_NB)]
        + [
            pltpu.VMEM_SHARED((ar, F1), jnp.float32),
            pltpu.SemaphoreType.DMA,
            pltpu.SemaphoreType.DMA,
        ],
    )
    def prop(ts, srcg, dstg, zrow, out, src_v, dst_v, *rest):
        locb = rest[:_NB]
        rowsb = rest[_NB:2 * _NB]
        acc, gsem, ssem = rest[2 * _NB:]
        c = lax.axis_index("c")
        s = lax.axis_index("s")
        w = c * _NS + s
        pltpu.sync_copy(srcg.at[w], src_v)
        pltpu.sync_copy(dstg.at[w], dst_v)

        for p in range(passes):
            base = p * rng
            plen = lens[p]
            @pl.when(s < ztiles)
            def _():
                pltpu.sync_copy(zrow, acc.at[pl.ds(s * zpt, zpt)])
            plsc.subcore_barrier()

            for b in range(_GA):
                pltpu.async_copy(ts.at[src_v.at[b]], rowsb[b], gsem)

            def outer(j, carry):
                for b in range(_NB):
                    jj = j * _NB + b
                    pltpu.make_async_copy(
                        ts.at[src_v.at[jj]], rowsb[b], gsem).wait()
                    for t in range(_K // 16):
                        dv = dst_v[jj, pl.ds(t * 16, 16)]
                        loc = dv - base
                        ok = (loc >= 0) & (loc < plen)
                        locb[b][pl.ds(t * 16, 16)] = jnp.where(
                            ok, loc, rng + t * 8)
                    pltpu.async_copy(
                        rowsb[b], acc.at[locb[b]], ssem, add=True)
                    bd = (b - _GA) % _NB
                    @pl.when(jj >= _GA)
                    def _():
                        pltpu.make_async_copy(
                            rowsb[bd], acc.at[locb[bd]], ssem).wait()
                    bg = (b + _GA) % _NB
                    @pl.when(jj + _GA < nch)
                    def _():
                        pltpu.async_copy(
                            ts.at[src_v.at[jj + _GA]], rowsb[bg], gsem)
                return carry

            lax.fori_loop(0, nch // _NB, outer, 0)
            for q in range(_GA):
                bd = (nch - _GA + q) % _NB
                pltpu.make_async_copy(rowsb[bd], acc.at[locb[bd]], ssem).wait()
            plsc.subcore_barrier()
            wtiles = _NS
            wpt = plen // wtiles
            while wpt % 8:
                wtiles //= 2
                wpt = plen // wtiles

            @pl.when(s < wtiles)
            def _():
                r0 = s * wpt
                pltpu.sync_copy(acc.at[pl.ds(r0, wpt)],
                                out.at[c, pl.ds(base + r0, wpt)])
            plsc.subcore_barrier()

    return prop


def _tc1(d0, d1, x):
    def body(d0_ref, d1_ref, x_ref, xs_ref, dinv_ref):
        deg = d0_ref[...][:, 0:1] + d1_ref[...][:, 0:1] + 1.0
        dinv = lax.rsqrt(deg)
        xs_ref[...] = x_ref[...] * dinv
        dinv_ref[...] = jnp.broadcast_to(dinv, (_BN, _DW))

    return pl.pallas_call(
        body,
        grid=(NP // _BN,),
        in_specs=[
            pl.BlockSpec((_BN, 4), lambda i: (i, 0)),
            pl.BlockSpec((_BN, 4), lambda i: (i, 0)),
            pl.BlockSpec((_BN, F1), lambda i: (i, 0)),
        ],
        out_specs=[
            pl.BlockSpec((_BN, F1), lambda i: (i, 0)),
            pl.BlockSpec((_BN, _DW), lambda i: (i, 0)),
        ],
        out_shape=[
            jax.ShapeDtypeStruct((NP, F1), jnp.float32),
            jax.ShapeDtypeStruct((NP, _DW), jnp.float32),
        ],
    )(d0, d1, x)


def _tc2(p1, xs, dinv, b1, w1):
    def body(p1_ref, xs_ref, dinv_ref, b1_ref, w1_ref, as_ref):
        di = dinv_ref[...][:, 0:1]
        px = (p1_ref[0] + p1_ref[1] + xs_ref[...]) * di
        h = jnp.dot(px, w1_ref[...], preferred_element_type=jnp.float32)
        as_ref[...] = jnp.maximum(h + b1_ref[...], 0.0) * di

    return pl.pallas_call(
        body,
        grid=(NP // _BN,),
        in_specs=[
            pl.BlockSpec((_NC, _BN, F1), lambda i: (0, i, 0)),
            pl.BlockSpec((_BN, F1), lambda i: (i, 0)),
            pl.BlockSpec((_BN, _DW), lambda i: (i, 0)),
            pl.BlockSpec((1, F1), lambda i: (0, 0)),
            pl.BlockSpec((F1, F1), lambda i: (0, 0)),
        ],
        out_specs=pl.BlockSpec((_BN, F1), lambda i: (i, 0)),
        out_shape=jax.ShapeDtypeStruct((NP, F1), jnp.float32),
    )(p1, xs, dinv, b1.reshape(1, F1), w1)


def _tc3(p2, as_, dinv, b2p, w2p):
    def body(p2_ref, as_ref, dinv_ref, b2p_ref, w2p_ref, out_ref):
        di = dinv_ref[...][:, 0:1]
        pa = (p2_ref[0] + p2_ref[1] + as_ref[...]) * di
        z = jnp.dot(pa, w2p_ref[...],
                    preferred_element_type=jnp.float32) + b2p_ref[...]
        zc = z[:, :C]
        m = jnp.max(zc, axis=1, keepdims=True)
        lse = jnp.log(jnp.sum(jnp.exp(zc - m), axis=1, keepdims=True)) + m
        out_ref[...] = zc - lse

    return pl.pallas_call(
        body,
        grid=(NP // _BN,),
        in_specs=[
            pl.BlockSpec((_NC, _BN, F1), lambda i: (0, i, 0)),
            pl.BlockSpec((_BN, F1), lambda i: (i, 0)),
            pl.BlockSpec((_BN, _DW), lambda i: (i, 0)),
            pl.BlockSpec((1, F2), lambda i: (0, 0)),
            pl.BlockSpec((F1, F2), lambda i: (0, 0)),
        ],
        out_specs=pl.BlockSpec((_BN, C), lambda i: (i, 0)),
        out_shape=jax.ShapeDtypeStruct((NP, C), jnp.float32),
    )(p2, as_, dinv, b2p.reshape(1, F2), w2p)


def kernel(x, edge_index, W1, b1, W2, b2):
    e = edge_index.shape[1]
    nch = -(-e // (_NW * _K * _NB)) * _NB
    epad = _NW * _K * nch - e
    srcf = jnp.concatenate([edge_index[0], jnp.zeros((epad,), jnp.int32)])
    dstf = jnp.concatenate([edge_index[1], jnp.full((epad,), N, jnp.int32)])
    src32 = srcf.reshape(_NW, nch, _K)
    dst32 = dstf.reshape(_NW, nch, _K)
    zrow = jnp.zeros((_geom(NP, 3, _GR)[4], F1), jnp.float32)
    zrowd = jnp.zeros((_geom(_DR, 1, 0)[4], F1), jnp.float32)
    pats = jnp.repeat(jnp.eye(32, dtype=jnp.float32), 4, axis=1)
    dmod = (dstf & 31).reshape(_NW, nch, _K)
    ddiv = (dstf >> 5).reshape(_NW, nch, _K)
    w2p = jnp.pad(W2, ((0, 0), (0, F2 - C)))
    b2p = jnp.pad(b2, (0, F2 - C))
    xp = jnp.pad(x, ((0, NP - N), (0, 0)))

    degp = _make_prop(nch, 1, _DR, 0)(pats, dmod, ddiv, zrowd)
    xs, dinv = _tc1(degp[0].reshape(NP, 4), degp[1].reshape(NP, 4), xp)
    prop = _make_prop(nch, 3)
    p1 = prop(xs, src32, dst32, zrow)
    as_ = _tc2(p1, xs, dinv, b1, W1)
    p2 = prop(as_, src32, dst32, zrow)
    return _tc3(p2, as_, dinv, b2p, w2p)[:N]


# speedup vs baseline: 1.7104x; 1.7104x over previous
import functools

import jax
import jax.numpy as jnp
from jax import lax
from jax.experimental import pallas as pl
from jax.experimental.pallas import tpu as pltpu
from jax.experimental.pallas import tpu_sc as plsc

N = 10000
NP = 10240
F1 = 128
C = 40
F2 = 48

_NC, _NS = 2, 16
_NW = _NC * _NS
_K = 80
_GR = 128
_NB = 2
_GA = 1
_DW = 16
_DR = NP // 32
_BN = 2048


def _geom(rows, passes, gr):
    if passes == 1 and gr == 0:
        rng = rows
    else:
        rng = -(-rows // (passes * 128)) * 128
    lens = [min(rng, rows - i * rng) for i in range(passes)]
    ar = rng + gr
    ztiles = _NS
    zpt = ar // ztiles
    while zpt % 8:
        ztiles //= 2
        zpt = ar // ztiles
    return rng, lens, ar, ztiles, zpt


def _make_prop(nch, passes, rows=NP, gr=_GR):
    rng, lens, ar, ztiles, zpt = _geom(rows, passes, gr)
    mesh = plsc.VectorSubcoreMesh(core_axis_name="c", subcore_axis_name="s")

    @functools.partial(
        pl.kernel,
        out_type=jax.ShapeDtypeStruct((_NC, rows, F1), jnp.float32),
        mesh=mesh,
        scratch_types=[
            pltpu.VMEM((nch, _K), jnp.int32),
            pltpu.VMEM((nch, _K), jnp.int32),
        ] + [pltpu.VMEM((_K,), jnp.int32) for _ in range(_NB)]
        + [pltpu.VMEM((_K, F1), jnp.float32) for _ in range(_NB)]
        + [
            pltpu.VMEM_SHARED((ar, F1), jnp.float32),
            pltpu.SemaphoreType.DMA,
            pltpu.SemaphoreType.DMA,
        ],
    )
    def prop(ts, srcg, dstg, zrow, out, src_v, dst_v, *rest):
        locb = rest[:_NB]
        rowsb = rest[_NB:2 * _NB]
        acc, gsem, ssem = rest[2 * _NB:]
        c = lax.axis_index("c")
        s = lax.axis_index("s")
        w = c * _NS + s
        pltpu.sync_copy(srcg.at[w], src_v)
        pltpu.sync_copy(dstg.at[w], dst_v)

        for p in range(passes):
            base = p * rng
            plen = lens[p]
            @pl.when(s < ztiles)
            def _():
                pltpu.sync_copy(zrow, acc.at[pl.ds(s * zpt, zpt)])
            plsc.subcore_barrier()

            for b in range(_GA):
                pltpu.async_copy(ts.at[src_v.at[b]], rowsb[b], gsem)

            def outer(j, carry):
                for b in range(_NB):
                    jj = j * _NB + b
                    pltpu.make_async_copy(
                        ts.at[src_v.at[jj]], rowsb[b], gsem).wait()
                    bg = (b + _GA) % _NB
                    @pl.when(jj + _GA < nch)
                    def _():
                        pltpu.async_copy(
                            ts.at[src_v.at[jj + _GA]], rowsb[bg], gsem)
                    for t in range(_K // 16):
                        dv = dst_v[jj, pl.ds(t * 16, 16)]
                        loc = dv - base
                        ok = (loc >= 0) & (loc < plen)
                        locb[b][pl.ds(t * 16, 16)] = jnp.where(
                            ok, loc, rng + t * 8)
                    pltpu.sync_copy(rowsb[b], acc.at[locb[b]], add=True)
                return carry

            lax.fori_loop(0, nch // _NB, outer, 0)
            plsc.subcore_barrier()
            wtiles = _NS
            wpt = plen // wtiles
            while wpt % 8:
                wtiles //= 2
                wpt = plen // wtiles

            @pl.when(s < wtiles)
            def _():
                r0 = s * wpt
                pltpu.sync_copy(acc.at[pl.ds(r0, wpt)],
                                out.at[c, pl.ds(base + r0, wpt)])
            plsc.subcore_barrier()

    return prop


def _tc1(d0, d1, x):
    def body(d0_ref, d1_ref, x_ref, xs_ref, dinv_ref):
        deg = d0_ref[...][:, 0:1] + d1_ref[...][:, 0:1] + 1.0
        dinv = lax.rsqrt(deg)
        xs_ref[...] = x_ref[...] * dinv
        dinv_ref[...] = jnp.broadcast_to(dinv, (_BN, _DW))

    return pl.pallas_call(
        body,
        grid=(NP // _BN,),
        in_specs=[
            pl.BlockSpec((_BN, 4), lambda i: (i, 0)),
            pl.BlockSpec((_BN, 4), lambda i: (i, 0)),
            pl.BlockSpec((_BN, F1), lambda i: (i, 0)),
        ],
        out_specs=[
            pl.BlockSpec((_BN, F1), lambda i: (i, 0)),
            pl.BlockSpec((_BN, _DW), lambda i: (i, 0)),
        ],
        out_shape=[
            jax.ShapeDtypeStruct((NP, F1), jnp.float32),
            jax.ShapeDtypeStruct((NP, _DW), jnp.float32),
        ],
    )(d0, d1, x)


def _tc2(p1, xs, dinv, b1, w1):
    def body(p1_ref, xs_ref, dinv_ref, b1_ref, w1_ref, as_ref):
        di = dinv_ref[...][:, 0:1]
        px = (p1_ref[0] + p1_ref[1] + xs_ref[...]) * di
        h = jnp.dot(px, w1_ref[...], preferred_element_type=jnp.float32)
        as_ref[...] = jnp.maximum(h + b1_ref[...], 0.0) * di

    return pl.pallas_call(
        body,
        grid=(NP // _BN,),
        in_specs=[
            pl.BlockSpec((_NC, _BN, F1), lambda i: (0, i, 0)),
            pl.BlockSpec((_BN, F1), lambda i: (i, 0)),
            pl.BlockSpec((_BN, _DW), lambda i: (i, 0)),
            pl.BlockSpec((1, F1), lambda i: (0, 0)),
            pl.BlockSpec((F1, F1), lambda i: (0, 0)),
        ],
        out_specs=pl.BlockSpec((_BN, F1), lambda i: (i, 0)),
        out_shape=jax.ShapeDtypeStruct((NP, F1), jnp.float32),
    )(p1, xs, dinv, b1.reshape(1, F1), w1)


def _tc3(p2, as_, dinv, b2p, w2p):
    def body(p2_ref, as_ref, dinv_ref, b2p_ref, w2p_ref, out_ref):
        di = dinv_ref[...][:, 0:1]
        pa = (p2_ref[0] + p2_ref[1] + as_ref[...]) * di
        z = jnp.dot(pa, w2p_ref[...],
                    preferred_element_type=jnp.float32) + b2p_ref[...]
        zc = z[:, :C]
        m = jnp.max(zc, axis=1, keepdims=True)
        lse = jnp.log(jnp.sum(jnp.exp(zc - m), axis=1, keepdims=True)) + m
        out_ref[...] = zc - lse

    return pl.pallas_call(
        body,
        grid=(NP // _BN,),
        in_specs=[
            pl.BlockSpec((_NC, _BN, F1), lambda i: (0, i, 0)),
            pl.BlockSpec((_BN, F1), lambda i: (i, 0)),
            pl.BlockSpec((_BN, _DW), lambda i: (i, 0)),
            pl.BlockSpec((1, F2), lambda i: (0, 0)),
            pl.BlockSpec((F1, F2), lambda i: (0, 0)),
        ],
        out_specs=pl.BlockSpec((_BN, C), lambda i: (i, 0)),
        out_shape=jax.ShapeDtypeStruct((NP, C), jnp.float32),
    )(p2, as_, dinv, b2p.reshape(1, F2), w2p)


def kernel(x, edge_index, W1, b1, W2, b2):
    e = edge_index.shape[1]
    nch = -(-e // (_NW * _K * _NB)) * _NB
    epad = _NW * _K * nch - e
    srcf = jnp.concatenate([edge_index[0], jnp.zeros((epad,), jnp.int32)])
    dstf = jnp.concatenate([edge_index[1], jnp.full((epad,), N, jnp.int32)])
    src32 = srcf.reshape(_NW, nch, _K)
    dst32 = dstf.reshape(_NW, nch, _K)
    zrow = jnp.zeros((_geom(NP, 3, _GR)[4], F1), jnp.float32)
    zrowd = jnp.zeros((_geom(_DR, 1, 0)[4], F1), jnp.float32)
    pats = jnp.repeat(jnp.eye(32, dtype=jnp.float32), 4, axis=1)
    dmod = (dstf & 31).reshape(_NW, nch, _K)
    ddiv = (dstf >> 5).reshape(_NW, nch, _K)
    w2p = jnp.pad(W2, ((0, 0), (0, F2 - C)))
    b2p = jnp.pad(b2, (0, F2 - C))
    xp = jnp.pad(x, ((0, NP - N), (0, 0)))

    degp = _make_prop(nch, 1, _DR, 0)(pats, dmod, ddiv, zrowd)
    xs, dinv = _tc1(degp[0].reshape(NP, 4), degp[1].reshape(NP, 4), xp)
    prop = _make_prop(nch, 3)
    p1 = prop(xs, src32, dst32, zrow)
    as_ = _tc2(p1, xs, dinv, b1, W1)
    p2 = prop(as_, src32, dst32, zrow)
    return _tc3(p2, as_, dinv, b2p, w2p)[:N]


# speedup vs baseline: 2.2497x; 1.3153x over previous
import functools

import jax
import jax.numpy as jnp
from jax import lax
from jax.experimental import pallas as pl
from jax.experimental.pallas import tpu as pltpu
from jax.experimental.pallas import tpu_sc as plsc

N = 10000
NP = 10240
F1 = 128
C = 40
F2 = 48

_NC, _NS = 2, 16
_NW = _NC * _NS
_K = 80
_GR = 128
_NB = 2
_GA = 1
_DW = 16
_DR = NP // 32
_BN = 2048


def _geom(rows, passes, gr):
    if passes == 1 and gr == 0:
        rng = rows
    else:
        rng = -(-rows // (passes * 128)) * 128
    lens = [min(rng, rows - i * rng) for i in range(passes)]
    ar = rng + gr
    ztiles = _NS
    zpt = ar // ztiles
    while zpt % 8:
        ztiles //= 2
        zpt = ar // ztiles
    return rng, lens, ar, ztiles, zpt


def _make_prop(nch, passes, rows=NP, gr=_GR):
    rng, lens, ar, ztiles, zpt = _geom(rows, passes, gr)
    mesh = plsc.VectorSubcoreMesh(core_axis_name="c", subcore_axis_name="s")

    @functools.partial(
        pl.kernel,
        out_type=jax.ShapeDtypeStruct((_NC, rows, F1), jnp.float32),
        mesh=mesh,
        scratch_types=[
            pltpu.VMEM((nch, _K), jnp.int32),
            pltpu.VMEM((nch, _K), jnp.int32),
        ] + [pltpu.VMEM((_K,), jnp.int32) for _ in range(_NB)]
        + [pltpu.VMEM((_K, F1), jnp.float32) for _ in range(_NB)]
        + [
            pltpu.VMEM_SHARED((ar, F1), jnp.float32),
            pltpu.SemaphoreType.DMA,
            pltpu.SemaphoreType.DMA,
        ],
    )
    def prop(ts, srcg, dstg, zrow, out, src_v, dst_v, *rest):
        locb = rest[:_NB]
        rowsb = rest[_NB:2 * _NB]
        acc, gsem, ssem = rest[2 * _NB:]
        c = lax.axis_index("c")
        s = lax.axis_index("s")
        w = c * _NS + s
        pltpu.sync_copy(srcg.at[w], src_v)
        pltpu.sync_copy(dstg.at[w], dst_v)

        for p in range(passes):
            base = p * rng
            plen = lens[p]
            @pl.when(s < ztiles)
            def _():
                pltpu.sync_copy(zrow, acc.at[pl.ds(s * zpt, zpt)])
            plsc.subcore_barrier()

            for b in range(_GA):
                pltpu.async_copy(ts.at[src_v.at[b]], rowsb[b], gsem)

            def outer(j, carry):
                for b in range(_NB):
                    jj = j * _NB + b
                    pltpu.make_async_copy(
                        ts.at[src_v.at[jj]], rowsb[b], gsem).wait()
                    bg = (b + _GA) % _NB
                    @pl.when(jj + _GA < nch)
                    def _():
                        pltpu.async_copy(
                            ts.at[src_v.at[jj + _GA]], rowsb[bg], gsem)
                    for t in range(_K // 16):
                        dv = dst_v[jj, pl.ds(t * 16, 16)]
                        loc = dv - base
                        ok = (loc >= 0) & (loc < plen)
                        locb[b][pl.ds(t * 16, 16)] = jnp.where(
                            ok, loc, rng + t * 8)
                    pltpu.sync_copy(rowsb[b], acc.at[locb[b]], add=True)
                return carry

            lax.fori_loop(0, nch // _NB, outer, 0)
            plsc.subcore_barrier()
            wtiles = _NS
            wpt = plen // wtiles
            while wpt % 8:
                wtiles //= 2
                wpt = plen // wtiles

            @pl.when(s < wtiles)
            def _():
                r0 = s * wpt
                pltpu.sync_copy(acc.at[pl.ds(r0, wpt)],
                                out.at[c, pl.ds(base + r0, wpt)])
            plsc.subcore_barrier()

    return prop


def _tc1(d0, d1, x):
    def body(d0_ref, d1_ref, x_ref, xs_ref, dinv_ref):
        deg = d0_ref[...][:, 0:1] + d1_ref[...][:, 0:1] + 1.0
        dinv = lax.rsqrt(deg)
        xs_ref[...] = x_ref[...] * dinv
        dinv_ref[...] = jnp.broadcast_to(dinv, (_BN, _DW))

    return pl.pallas_call(
        body,
        grid=(NP // _BN,),
        in_specs=[
            pl.BlockSpec((_BN, 4), lambda i: (i, 0)),
            pl.BlockSpec((_BN, 4), lambda i: (i, 0)),
            pl.BlockSpec((_BN, F1), lambda i: (i, 0)),
        ],
        out_specs=[
            pl.BlockSpec((_BN, F1), lambda i: (i, 0)),
            pl.BlockSpec((_BN, _DW), lambda i: (i, 0)),
        ],
        out_shape=[
            jax.ShapeDtypeStruct((NP, F1), jnp.float32),
            jax.ShapeDtypeStruct((NP, _DW), jnp.float32),
        ],
    )(d0, d1, x)


def _tc2(p1, xs, dinv, b1, w1):
    def body(p1_ref, xs_ref, dinv_ref, b1_ref, w1_ref, as_ref):
        di = dinv_ref[...][:, 0:1]
        px = (p1_ref[0] + p1_ref[1] + xs_ref[...]) * di
        h = jnp.dot(px, w1_ref[...], preferred_element_type=jnp.float32)
        as_ref[...] = jnp.maximum(h + b1_ref[...], 0.0) * di

    return pl.pallas_call(
        body,
        grid=(NP // _BN,),
        in_specs=[
            pl.BlockSpec((_NC, _BN, F1), lambda i: (0, i, 0)),
            pl.BlockSpec((_BN, F1), lambda i: (i, 0)),
            pl.BlockSpec((_BN, _DW), lambda i: (i, 0)),
            pl.BlockSpec((1, F1), lambda i: (0, 0)),
            pl.BlockSpec((F1, F1), lambda i: (0, 0)),
        ],
        out_specs=pl.BlockSpec((_BN, F1), lambda i: (i, 0)),
        out_shape=jax.ShapeDtypeStruct((NP, F1), jnp.float32),
    )(p1, xs, dinv, b1.reshape(1, F1), w1)


def _tc3(p2, as_, dinv, b2p, w2p):
    def body(p2_ref, as_ref, dinv_ref, b2p_ref, w2p_ref, out_ref):
        di = dinv_ref[...][:, 0:1]
        pa = (p2_ref[0] + p2_ref[1] + as_ref[...]) * di
        z = jnp.dot(pa, w2p_ref[...],
                    preferred_element_type=jnp.float32) + b2p_ref[...]
        zc = z[:, :C]
        m = jnp.max(zc, axis=1, keepdims=True)
        lse = jnp.log(jnp.sum(jnp.exp(zc - m), axis=1, keepdims=True)) + m
        out_ref[...] = zc - lse

    return pl.pallas_call(
        body,
        grid=(NP // _BN,),
        in_specs=[
            pl.BlockSpec((_NC, _BN, F1), lambda i: (0, i, 0)),
            pl.BlockSpec((_BN, F1), lambda i: (i, 0)),
            pl.BlockSpec((_BN, _DW), lambda i: (i, 0)),
            pl.BlockSpec((1, F2), lambda i: (0, 0)),
            pl.BlockSpec((F1, F2), lambda i: (0, 0)),
        ],
        out_specs=pl.BlockSpec((_BN, C), lambda i: (i, 0)),
        out_shape=jax.ShapeDtypeStruct((NP, C), jnp.float32),
    )(p2, as_, dinv, b2p.reshape(1, F2), w2p)


def kernel(x, edge_index, W1, b1, W2, b2):
    e = edge_index.shape[1]
    nch = -(-e // (_NW * _K * _NB)) * _NB
    epad = _NW * _K * nch - e
    srcf = jnp.concatenate([edge_index[0], jnp.zeros((epad,), jnp.int32)])
    dstf = jnp.concatenate([edge_index[1], jnp.full((epad,), N, jnp.int32)])
    src32 = srcf.reshape(_NW, nch, _K)
    dst32 = dstf.reshape(_NW, nch, _K)
    zrow = jnp.zeros((_geom(NP, 2, _GR)[4], F1), jnp.float32)
    zrowd = jnp.zeros((_geom(_DR, 1, 0)[4], F1), jnp.float32)
    pats = jnp.repeat(jnp.eye(32, dtype=jnp.float32), 4, axis=1)
    dmod = (dstf & 31).reshape(_NW, nch, _K)
    ddiv = (dstf >> 5).reshape(_NW, nch, _K)
    w2p = jnp.pad(W2, ((0, 0), (0, F2 - C)))
    b2p = jnp.pad(b2, (0, F2 - C))
    xp = jnp.pad(x, ((0, NP - N), (0, 0)))

    degp = _make_prop(nch, 1, _DR, 0)(pats, dmod, ddiv, zrowd)
    xs, dinv = _tc1(degp[0].reshape(NP, 4), degp[1].reshape(NP, 4), xp)
    prop = _make_prop(nch, 2)
    p1 = prop(xs, src32, dst32, zrow)
    as_ = _tc2(p1, xs, dinv, b1, W1)
    p2 = prop(as_, src32, dst32, zrow)
    return _tc3(p2, as_, dinv, b2p, w2p)[:N]


# speedup vs baseline: 2.8440x; 1.2642x over previous
import functools

import jax
import jax.numpy as jnp
from jax import lax
from jax.experimental import pallas as pl
from jax.experimental.pallas import tpu as pltpu
from jax.experimental.pallas import tpu_sc as plsc

N = 10000
NP = 10240
F1 = 128
C = 40
F2 = 48

_NC, _NS = 2, 16
_NW = _NC * _NS
_K = 80
_GR = 128
_NB = 2
_GA = 1
_DW = 16
_DR = NP // 32
_BN = 2048


def _geom(rows, passes, gr):
    if passes == 1 and gr == 0:
        rng = rows
    else:
        rng = -(-rows // (passes * 128)) * 128
    lens = [min(rng, rows - i * rng) for i in range(passes)]
    ar = rng + gr
    ztiles = _NS
    zpt = ar // ztiles
    while zpt % 8:
        ztiles //= 2
        zpt = ar // ztiles
    return rng, lens, ar, ztiles, zpt


def _make_prop(nch, passes, rows=NP, gr=_GR):
    rng, lens, ar, ztiles, zpt = _geom(rows, passes, gr)
    mesh = plsc.VectorSubcoreMesh(core_axis_name="c", subcore_axis_name="s")

    @functools.partial(
        pl.kernel,
        out_type=jax.ShapeDtypeStruct((_NC, rows, F1), jnp.float32),
        mesh=mesh,
        scratch_types=[
            pltpu.VMEM((nch, _K), jnp.int32),
            pltpu.VMEM((nch, _K), jnp.int32),
        ] + [pltpu.VMEM((_K,), jnp.int32) for _ in range(_NB)]
        + [pltpu.VMEM((_K, F1), jnp.float32) for _ in range(_NB)]
        + [
            pltpu.VMEM_SHARED((ar, F1), jnp.float32),
            pltpu.SemaphoreType.DMA,
            pltpu.SemaphoreType.DMA,
        ],
    )
    def prop(ts, srcg, dstg, zrow, out, src_v, dst_v, *rest):
        locb = rest[:_NB]
        rowsb = rest[_NB:2 * _NB]
        acc, gsem, ssem = rest[2 * _NB:]
        c = lax.axis_index("c")
        s = lax.axis_index("s")
        w = c * _NS + s
        pltpu.sync_copy(srcg.at[w], src_v)
        pltpu.sync_copy(dstg.at[w], dst_v)

        for p in range(passes):
            base = p * rng
            plen = lens[p]
            @pl.when(s < ztiles)
            def _():
                pltpu.sync_copy(zrow, acc.at[pl.ds(s * zpt, zpt)])
            plsc.subcore_barrier()

            for b in range(_GA):
                pltpu.async_copy(ts.at[src_v.at[b]], rowsb[b], gsem)

            def outer(j, carry):
                for b in range(_NB):
                    jj = j * _NB + b
                    pltpu.make_async_copy(
                        ts.at[src_v.at[jj]], rowsb[b], gsem).wait()
                    bg = (b + _GA) % _NB
                    @pl.when(jj + _GA < nch)
                    def _():
                        pltpu.async_copy(
                            ts.at[src_v.at[jj + _GA]], rowsb[bg], gsem)
                    for t in range(_K // 16):
                        dv = dst_v[jj, pl.ds(t * 16, 16)]
                        loc = dv - base
                        ok = (loc >= 0) & (loc < plen)
                        locb[b][pl.ds(t * 16, 16)] = jnp.where(
                            ok, loc, rng + t * 8)
                    pltpu.sync_copy(rowsb[b], acc.at[locb[b]], add=True)
                return carry

            lax.fori_loop(0, nch // _NB, outer, 0)
            plsc.subcore_barrier()
            wtiles = _NS
            wpt = plen // wtiles
            while wpt % 8:
                wtiles //= 2
                wpt = plen // wtiles

            @pl.when(s < wtiles)
            def _():
                r0 = s * wpt
                pltpu.sync_copy(acc.at[pl.ds(r0, wpt)],
                                out.at[c, pl.ds(base + r0, wpt)])
            plsc.subcore_barrier()

    return prop


def _tc1(d0, d1, x):
    def body(d0_ref, d1_ref, x_ref, xs_ref, dinv_ref):
        deg = d0_ref[...][:, 0:1] + d1_ref[...][:, 0:1] + 1.0
        dinv = lax.rsqrt(deg)
        xs_ref[...] = x_ref[...] * dinv
        dinv_ref[...] = jnp.broadcast_to(dinv, (_BN, _DW))

    return pl.pallas_call(
        body,
        grid=(NP // _BN,),
        in_specs=[
            pl.BlockSpec((_BN, 4), lambda i: (i, 0)),
            pl.BlockSpec((_BN, 4), lambda i: (i, 0)),
            pl.BlockSpec((_BN, F1), lambda i: (i, 0)),
        ],
        out_specs=[
            pl.BlockSpec((_BN, F1), lambda i: (i, 0)),
            pl.BlockSpec((_BN, _DW), lambda i: (i, 0)),
        ],
        out_shape=[
            jax.ShapeDtypeStruct((NP, F1), jnp.float32),
            jax.ShapeDtypeStruct((NP, _DW), jnp.float32),
        ],
    )(d0, d1, x)


def _tc2(p1, xs, dinv, b1, w1):
    def body(p1_ref, xs_ref, dinv_ref, b1_ref, w1_ref, as_ref):
        di = dinv_ref[...][:, 0:1]
        px = (p1_ref[0] + p1_ref[1] + xs_ref[...]) * di
        h = jnp.dot(px, w1_ref[...], preferred_element_type=jnp.float32)
        as_ref[...] = jnp.maximum(h + b1_ref[...], 0.0) * di

    return pl.pallas_call(
        body,
        grid=(NP // _BN,),
        in_specs=[
            pl.BlockSpec((_NC, _BN, F1), lambda i: (0, i, 0)),
            pl.BlockSpec((_BN, F1), lambda i: (i, 0)),
            pl.BlockSpec((_BN, _DW), lambda i: (i, 0)),
            pl.BlockSpec((1, F1), lambda i: (0, 0)),
            pl.BlockSpec((F1, F1), lambda i: (0, 0)),
        ],
        out_specs=pl.BlockSpec((_BN, F1), lambda i: (i, 0)),
        out_shape=jax.ShapeDtypeStruct((NP, F1), jnp.float32),
    )(p1, xs, dinv, b1.reshape(1, F1), w1)


def _tc3(p2, as_, dinv, b2p, w2p):
    def body(p2_ref, as_ref, dinv_ref, b2p_ref, w2p_ref, out_ref):
        di = dinv_ref[...][:, 0:1]
        pa = (p2_ref[0] + p2_ref[1] + as_ref[...]) * di
        z = jnp.dot(pa, w2p_ref[...],
                    preferred_element_type=jnp.float32) + b2p_ref[...]
        zc = z[:, :C]
        m = jnp.max(zc, axis=1, keepdims=True)
        lse = jnp.log(jnp.sum(jnp.exp(zc - m), axis=1, keepdims=True)) + m
        out_ref[...] = zc - lse

    return pl.pallas_call(
        body,
        grid=(NP // _BN,),
        in_specs=[
            pl.BlockSpec((_NC, _BN, F1), lambda i: (0, i, 0)),
            pl.BlockSpec((_BN, F1), lambda i: (i, 0)),
            pl.BlockSpec((_BN, _DW), lambda i: (i, 0)),
            pl.BlockSpec((1, F2), lambda i: (0, 0)),
            pl.BlockSpec((F1, F2), lambda i: (0, 0)),
        ],
        out_specs=pl.BlockSpec((_BN, C), lambda i: (i, 0)),
        out_shape=jax.ShapeDtypeStruct((NP, C), jnp.float32),
    )(p2, as_, dinv, b2p.reshape(1, F2), w2p)


def kernel(x, edge_index, W1, b1, W2, b2):
    e = edge_index.shape[1]
    nch = -(-e // (_NW * _K * _NB)) * _NB
    epad = _NW * _K * nch - e
    srcf = jnp.concatenate([edge_index[0], jnp.zeros((epad,), jnp.int32)])
    dstf = jnp.concatenate([edge_index[1], jnp.full((epad,), N, jnp.int32)])
    src32 = srcf.reshape(_NW, nch, _K)
    dst32 = dstf.reshape(_NW, nch, _K)
    zrow = jnp.zeros((_geom(NP, 2, _GR)[4], F1), jnp.float32)
    zrowd = jnp.zeros((_geom(_DR, 1, 0)[4], F1), jnp.float32)
    pats = jnp.tile(jnp.repeat(jnp.eye(32, dtype=jnp.float32), 4, axis=1),
                    (8, 1))
    stripe = (jnp.arange(dstf.shape[0], dtype=jnp.int32) & 7) << 5
    dmod = ((dstf & 31) + stripe).reshape(_NW, nch, _K)
    ddiv = (dstf >> 5).reshape(_NW, nch, _K)
    w2p = jnp.pad(W2, ((0, 0), (0, F2 - C)))
    b2p = jnp.pad(b2, (0, F2 - C))
    xp = jnp.pad(x, ((0, NP - N), (0, 0)))

    degp = _make_prop(nch, 1, _DR, 0)(pats, dmod, ddiv, zrowd)
    xs, dinv = _tc1(degp[0].reshape(NP, 4), degp[1].reshape(NP, 4), xp)
    prop = _make_prop(nch, 2)
    p1 = prop(xs, src32, dst32, zrow)
    as_ = _tc2(p1, xs, dinv, b1, W1)
    p2 = prop(as_, src32, dst32, zrow)
    return _tc3(p2, as_, dinv, b2p, w2p)[:N]


# speedup vs baseline: 2.8872x; 1.0152x over previous
import functools

import jax
import jax.numpy as jnp
from jax import lax
from jax.experimental import pallas as pl
from jax.experimental.pallas import tpu as pltpu
from jax.experimental.pallas import tpu_sc as plsc

N = 10000
NP = 10240
F1 = 128
C = 40
F2 = 48

_NC, _NS = 2, 16
_NW = _NC * _NS
_K = 80
_GR = 128
_NB = 2
_GA = 1
_DW = 16
_DR = NP // 32
_BN = 2048


def _geom(rows, passes, gr):
    if passes == 1 and gr == 0:
        rng = rows
    else:
        rng = -(-rows // (passes * 128)) * 128
    lens = [min(rng, rows - i * rng) for i in range(passes)]
    ar = rng + gr
    ztiles = _NS
    zpt = ar // ztiles
    while zpt % 8:
        ztiles //= 2
        zpt = ar // ztiles
    return rng, lens, ar, ztiles, zpt


def _make_prop(nch, passes, rows=NP, gr=_GR):
    rng, lens, ar, ztiles, zpt = _geom(rows, passes, gr)
    mesh = plsc.VectorSubcoreMesh(core_axis_name="c", subcore_axis_name="s")

    @functools.partial(
        pl.kernel,
        out_type=jax.ShapeDtypeStruct((_NC, rows, F1), jnp.float32),
        mesh=mesh,
        scratch_types=[
            pltpu.VMEM((nch, _K), jnp.int32),
            pltpu.VMEM((nch, _K), jnp.int32),
        ] + [pltpu.VMEM((_K,), jnp.int32) for _ in range(_NB)]
        + [pltpu.VMEM((_K, F1), jnp.float32) for _ in range(_NB)]
        + [
            pltpu.VMEM_SHARED((ar, F1), jnp.float32),
            pltpu.SemaphoreType.DMA,
            pltpu.SemaphoreType.DMA,
        ],
    )
    def prop(ts, srcg, dstg, zrow, out, src_v, dst_v, *rest):
        locb = rest[:_NB]
        rowsb = rest[_NB:2 * _NB]
        acc, gsem, ssem = rest[2 * _NB:]
        c = lax.axis_index("c")
        s = lax.axis_index("s")
        w = c * _NS + s
        pltpu.sync_copy(srcg.at[w], src_v)
        pltpu.sync_copy(dstg.at[w], dst_v)

        for p in range(passes):
            base = p * rng
            plen = lens[p]
            @pl.when(s < ztiles)
            def _():
                pltpu.sync_copy(zrow, acc.at[pl.ds(s * zpt, zpt)])
            plsc.subcore_barrier()

            for b in range(_GA):
                pltpu.async_copy(ts.at[src_v.at[b]], rowsb[b], gsem)

            def outer(j, carry):
                for b in range(_NB):
                    jj = j * _NB + b
                    pltpu.make_async_copy(
                        ts.at[src_v.at[jj]], rowsb[b], gsem).wait()
                    bg = (b + _GA) % _NB
                    @pl.when(jj + _GA < nch)
                    def _():
                        pltpu.async_copy(
                            ts.at[src_v.at[jj + _GA]], rowsb[bg], gsem)
                    for t in range(_K // 16):
                        dv = dst_v[jj, pl.ds(t * 16, 16)]
                        loc = dv - base
                        ok = (loc >= 0) & (loc < plen)
                        gid = rng + (((jj * 5 + t) & 15) << 3)
                        locb[b][pl.ds(t * 16, 16)] = jnp.where(ok, loc, gid)
                    pltpu.sync_copy(rowsb[b], acc.at[locb[b]], add=True)
                return carry

            lax.fori_loop(0, nch // _NB, outer, 0)
            plsc.subcore_barrier()
            wtiles = _NS
            wpt = plen // wtiles
            while wpt % 8:
                wtiles //= 2
                wpt = plen // wtiles

            @pl.when(s < wtiles)
            def _():
                r0 = s * wpt
                pltpu.sync_copy(acc.at[pl.ds(r0, wpt)],
                                out.at[c, pl.ds(base + r0, wpt)])
            plsc.subcore_barrier()

    return prop


def _tc1(d0, d1, x):
    def body(d0_ref, d1_ref, x_ref, xs_ref, dinv_ref):
        deg = d0_ref[...][:, 0:1] + d1_ref[...][:, 0:1] + 1.0
        dinv = lax.rsqrt(deg)
        xs_ref[...] = x_ref[...] * dinv
        dinv_ref[...] = jnp.broadcast_to(dinv, (_BN, _DW))

    return pl.pallas_call(
        body,
        grid=(NP // _BN,),
        in_specs=[
            pl.BlockSpec((_BN, 4), lambda i: (i, 0)),
            pl.BlockSpec((_BN, 4), lambda i: (i, 0)),
            pl.BlockSpec((_BN, F1), lambda i: (i, 0)),
        ],
        out_specs=[
            pl.BlockSpec((_BN, F1), lambda i: (i, 0)),
            pl.BlockSpec((_BN, _DW), lambda i: (i, 0)),
        ],
        out_shape=[
            jax.ShapeDtypeStruct((NP, F1), jnp.float32),
            jax.ShapeDtypeStruct((NP, _DW), jnp.float32),
        ],
    )(d0, d1, x)


def _tc2(p1, xs, dinv, b1, w1):
    def body(p1_ref, xs_ref, dinv_ref, b1_ref, w1_ref, as_ref):
        di = dinv_ref[...][:, 0:1]
        px = (p1_ref[0] + p1_ref[1] + xs_ref[...]) * di
        h = jnp.dot(px, w1_ref[...], preferred_element_type=jnp.float32)
        as_ref[...] = jnp.maximum(h + b1_ref[...], 0.0) * di

    return pl.pallas_call(
        body,
        grid=(NP // _BN,),
        in_specs=[
            pl.BlockSpec((_NC, _BN, F1), lambda i: (0, i, 0)),
            pl.BlockSpec((_BN, F1), lambda i: (i, 0)),
            pl.BlockSpec((_BN, _DW), lambda i: (i, 0)),
            pl.BlockSpec((1, F1), lambda i: (0, 0)),
            pl.BlockSpec((F1, F1), lambda i: (0, 0)),
        ],
        out_specs=pl.BlockSpec((_BN, F1), lambda i: (i, 0)),
        out_shape=jax.ShapeDtypeStruct((NP, F1), jnp.float32),
    )(p1, xs, dinv, b1.reshape(1, F1), w1)


def _tc3(p2, as_, dinv, b2p, w2p):
    def body(p2_ref, as_ref, dinv_ref, b2p_ref, w2p_ref, out_ref):
        di = dinv_ref[...][:, 0:1]
        pa = (p2_ref[0] + p2_ref[1] + as_ref[...]) * di
        z = jnp.dot(pa, w2p_ref[...],
                    preferred_element_type=jnp.float32) + b2p_ref[...]
        zc = z[:, :C]
        m = jnp.max(zc, axis=1, keepdims=True)
        lse = jnp.log(jnp.sum(jnp.exp(zc - m), axis=1, keepdims=True)) + m
        out_ref[...] = zc - lse

    return pl.pallas_call(
        body,
        grid=(NP // _BN,),
        in_specs=[
            pl.BlockSpec((_NC, _BN, F1), lambda i: (0, i, 0)),
            pl.BlockSpec((_BN, F1), lambda i: (i, 0)),
            pl.BlockSpec((_BN, _DW), lambda i: (i, 0)),
            pl.BlockSpec((1, F2), lambda i: (0, 0)),
            pl.BlockSpec((F1, F2), lambda i: (0, 0)),
        ],
        out_specs=pl.BlockSpec((_BN, C), lambda i: (i, 0)),
        out_shape=jax.ShapeDtypeStruct((NP, C), jnp.float32),
    )(p2, as_, dinv, b2p.reshape(1, F2), w2p)


def kernel(x, edge_index, W1, b1, W2, b2):
    e = edge_index.shape[1]
    nch = -(-e // (_NW * _K * _NB)) * _NB
    epad = _NW * _K * nch - e
    srcf = jnp.concatenate([edge_index[0], jnp.zeros((epad,), jnp.int32)])
    dstf = jnp.concatenate([edge_index[1], jnp.full((epad,), N, jnp.int32)])
    src32 = srcf.reshape(_NW, nch, _K)
    dst32 = dstf.reshape(_NW, nch, _K)
    zrow = jnp.zeros((_geom(NP, 2, _GR)[4], F1), jnp.float32)
    zrowd = jnp.zeros((_geom(_DR, 1, 0)[4], F1), jnp.float32)
    pats = jnp.tile(jnp.repeat(jnp.eye(32, dtype=jnp.float32), 4, axis=1),
                    (16, 1))
    stripe = (jnp.arange(dstf.shape[0], dtype=jnp.int32) & 15) << 5
    dmod = ((dstf & 31) + stripe).reshape(_NW, nch, _K)
    ddiv = (dstf >> 5).reshape(_NW, nch, _K)
    w2p = jnp.pad(W2, ((0, 0), (0, F2 - C)))
    b2p = jnp.pad(b2, (0, F2 - C))
    xp = jnp.pad(x, ((0, NP - N), (0, 0)))

    degp = _make_prop(nch, 1, _DR, 0)(pats, dmod, ddiv, zrowd)
    xs, dinv = _tc1(degp[0].reshape(NP, 4), degp[1].reshape(NP, 4), xp)
    prop = _make_prop(nch, 2)
    p1 = prop(xs, src32, dst32, zrow)
    as_ = _tc2(p1, xs, dinv, b1, W1)
    p2 = prop(as_, src32, dst32, zrow)
    return _tc3(p2, as_, dinv, b2p, w2p)[:N]
